# baseline (speedup 1.0000x reference)
import functools
import jax
import jax.numpy as jnp
from jax import lax
from jax.experimental import pallas as pl
from jax.experimental.pallas import tpu as pltpu
from jax.experimental.pallas import tpu_sc as plsc

B = 2048
H = 1024
HH = 512
E = 16
OUT = 16
EO = E * OUT
BH = B // 2

BB = 512
BQ = BH // (B // BB)
NC = 2
NS = 16
NW = NC * NS
L = 16
TPW = B // NW
NG = TPW // L


def _k1_kernel(q_ref, W1_ref, b1_ref, W2T_ref, b2_ref,
               x_ref, Wef_ref, logits_ref, y_ref):
    h = jnp.maximum(jnp.dot(q_ref[...], W1_ref[...],
                            preferred_element_type=jnp.float32) + b1_ref[...], 0.0)
    logits_ref[...] = lax.dot_general(
        h, W2T_ref[...], dimension_numbers=(((1,), (1,)), ((), ())),
        preferred_element_type=jnp.float32) + b2_ref[...]
    y_ref[...] = lax.dot_general(
        x_ref[...], Wef_ref[...],
        dimension_numbers=(((1,), (1,)), ((), ())),
        preferred_element_type=jnp.float32)


def _k1(q, W1, b1, W2T, b2, x, We_flat):
    return pl.pallas_call(
        _k1_kernel,
        grid=(B // BB,),
        in_specs=[
            pl.BlockSpec((BB, H), lambda i: (i, 0)),
            pl.BlockSpec((H, HH), lambda i: (0, 0)),
            pl.BlockSpec((HH,), lambda i: (0,)),
            pl.BlockSpec((E, HH), lambda i: (0, 0)),
            pl.BlockSpec((E,), lambda i: (0,)),
            pl.BlockSpec((BQ, H), lambda i: (i, 0)),
            pl.BlockSpec((EO, H), lambda i: (0, 0)),
        ],
        out_specs=[
            pl.BlockSpec((BB, E), lambda i: (i, 0)),
            pl.BlockSpec((BQ, EO), lambda i: (i, 0)),
        ],
        out_shape=[
            jax.ShapeDtypeStruct((B, E), jnp.float32),
            jax.ShapeDtypeStruct((BH, EO), jnp.float32),
        ],
    )(q, W1, b1, W2T, b2, x, We_flat)


def _k2_kernel(x_ref, Wef_ref, y_ref):
    y_ref[...] = lax.dot_general(
        x_ref[...], Wef_ref[...],
        dimension_numbers=(((1,), (1,)), ((), ())),
        preferred_element_type=jnp.float32)


def _k2(x, We_flat):
    nlo = BH // BB
    return pl.pallas_call(
        _k2_kernel,
        grid=(BH // BB,),
        in_specs=[
            pl.BlockSpec((BB, H), lambda i: (i + nlo, 0)),
            pl.BlockSpec((EO, H), lambda i: (0, 0)),
        ],
        out_specs=pl.BlockSpec((BB, EO), lambda i: (i, 0)),
        out_shape=jax.ShapeDtypeStruct((BH, EO), jnp.float32),
    )(x, We_flat)


def _k3_kernel(g_ref, y_lo_ref, y_hi_ref, beT_ref, outT_ref):
    je = lax.broadcasted_iota(jnp.int32, (E, EO), 1) // OUT
    ee = lax.broadcasted_iota(jnp.int32, (E, EO), 0)
    R = (je == ee).astype(jnp.float32)
    jo = lax.broadcasted_iota(jnp.int32, (OUT, EO), 1) % OUT
    oo = lax.broadcasted_iota(jnp.int32, (OUT, EO), 0)
    ST = (jo == oo).astype(jnp.float32)
    half = pl.program_id(0) // (BH // BK)
    y = jnp.where(half == 0, y_lo_ref[...], y_hi_ref[...])
    g = g_ref[...]
    gbig = jnp.dot(g, R, preferred_element_type=jnp.float32)
    outT_ref[...] = lax.dot_general(
        ST, gbig * y, dimension_numbers=(((1,), (1,)), ((), ())),
        preferred_element_type=jnp.float32) + lax.dot_general(
        beT_ref[...], g, dimension_numbers=(((1,), (1,)), ((), ())),
        preferred_element_type=jnp.float32)


BK = 512


def _k3(gates, y_lo, y_hi, beT):
    nlo = BH // BK
    return pl.pallas_call(
        _k3_kernel,
        grid=(B // BK,),
        in_specs=[
            pl.BlockSpec((BK, E), lambda i: (i, 0)),
            pl.BlockSpec((BK, EO), lambda i: (jnp.minimum(i, nlo - 1), 0)),
            pl.BlockSpec((BK, EO), lambda i: (jnp.maximum(i - nlo, 0), 0)),
            pl.BlockSpec((OUT, E), lambda i: (0, 0)),
        ],
        out_specs=pl.BlockSpec((OUT, BK), lambda i: (0, i)),
        out_shape=jax.ShapeDtypeStruct((OUT, B), jnp.float32),
    )(gates, y_lo, y_hi, beT)


def _sc_routing_body(logits_hbm, gates_hbm, lg_v, g_v):
    wid = lax.axis_index("s") * NC + lax.axis_index("c")
    base = wid * TPW
    pltpu.sync_copy(logits_hbm.at[pl.ds(base, TPW)], lg_v)

    lanes = lax.iota(jnp.int32, L)

    def group_body(g, _):
        rowidx = g * L + lanes
        def top2_body(e, carry):
            m0, m1, i0, i1 = carry
            v = plsc.load_gather(lg_v, [rowidx, jnp.full((L,), e, jnp.int32)])
            is0 = v > m0
            is1 = jnp.logical_and(jnp.logical_not(is0), v > m1)
            m1 = jnp.where(is0, m0, jnp.where(is1, v, m1))
            i1 = jnp.where(is0, i0, jnp.where(is1, e, i1))
            m0 = jnp.where(is0, v, m0)
            i0 = jnp.where(is0, e, i0)
            return m0, m1, i0, i1

        init = (jnp.full((L,), -jnp.inf, jnp.float32),
                jnp.full((L,), -jnp.inf, jnp.float32),
                jnp.zeros((L,), jnp.int32),
                jnp.zeros((L,), jnp.int32))
        m0, m1, i0, i1 = lax.fori_loop(0, E, top2_body, init)
        ex = jnp.exp(m1 - m0)
        g0 = 1.0 / (1.0 + ex)
        g1 = ex * g0

        def scatter_body(e, _):
            col = jnp.where(i0 == e, g0, 0.0) + jnp.where(i1 == e, g1, 0.0)
            plsc.store_scatter(g_v, [rowidx, jnp.full((L,), e, jnp.int32)], col)
            return 0

        lax.fori_loop(0, E, scatter_body, 0)
        return 0

    lax.fori_loop(0, NG, group_body, 0)

    pltpu.sync_copy(g_v, gates_hbm.at[pl.ds(base, TPW)])


_sc_routing = functools.partial(
    pl.kernel,
    mesh=plsc.VectorSubcoreMesh(core_axis_name="c", subcore_axis_name="s",
                                num_cores=NC, num_subcores=NS),
    compiler_params=pltpu.CompilerParams(needs_layout_passes=False),
    out_type=jax.ShapeDtypeStruct((B, E), jnp.float32),
    scratch_types=[
        pltpu.VMEM((TPW, E), jnp.float32),
        pltpu.VMEM((TPW, E), jnp.float32),
    ],
)(_sc_routing_body)


def kernel(query_repr, x, W1, b1, W2, b2, We, be):
    We_flat = We.reshape(EO, H)
    logits, y_lo = _k1(query_repr, W1, b1, W2.T, b2, x, We_flat)
    gates = _sc_routing(logits)
    y_hi = _k2(x, We_flat)
    return _k3(gates, y_lo, y_hi, be.T).T


# speedup vs baseline: 1.0218x; 1.0218x over previous
import functools
import jax
import jax.numpy as jnp
from jax import lax
from jax.experimental import pallas as pl
from jax.experimental.pallas import tpu as pltpu
from jax.experimental.pallas import tpu_sc as plsc

B = 2048
H = 1024
HH = 512
E = 16
OUT = 16
EO = E * OUT
BH = B // 2

BB = 512
BQ = BH // (B // BB)
NC = 2
NS = 16
NW = NC * NS
L = 16
TPW = B // NW
NG = TPW // L


def _k1_kernel(qa_ref, qb_ref, W1_ref, b1_ref, W2T_ref, b2_ref,
               x_ref, Wef_ref, logits_ref, y_ref):
    q = jnp.concatenate([qa_ref[...], qb_ref[...]], axis=0)
    h = jnp.maximum(jnp.dot(q, W1_ref[...],
                            preferred_element_type=jnp.float32) + b1_ref[...], 0.0)
    logits_ref[...] = lax.dot_general(
        h, W2T_ref[...], dimension_numbers=(((1,), (1,)), ((), ())),
        preferred_element_type=jnp.float32) + b2_ref[...]
    y_ref[...] = lax.dot_general(
        x_ref[...], Wef_ref[...],
        dimension_numbers=(((1,), (1,)), ((), ())),
        preferred_element_type=jnp.float32)


def _k1(q, W1, b1, W2T, b2, x, We_flat):
    return pl.pallas_call(
        _k1_kernel,
        grid=(B // BB,),
        in_specs=[
            pl.BlockSpec((BQ, H), lambda i: (2 * i, 0)),
            pl.BlockSpec((BQ, H), lambda i: (2 * i + 1, 0)),
            pl.BlockSpec((H, HH), lambda i: (0, 0)),
            pl.BlockSpec((HH,), lambda i: (0,)),
            pl.BlockSpec((E, HH), lambda i: (0, 0)),
            pl.BlockSpec((E,), lambda i: (0,)),
            pl.BlockSpec((BQ, H), lambda i: (i, 0)),
            pl.BlockSpec((EO, H), lambda i: (0, 0)),
        ],
        out_specs=[
            pl.BlockSpec((BB, E), lambda i: (i, 0)),
            pl.BlockSpec((BQ, EO), lambda i: (i, 0)),
        ],
        out_shape=[
            jax.ShapeDtypeStruct((B, E), jnp.float32),
            jax.ShapeDtypeStruct((BH, EO), jnp.float32),
        ],
    )(q, q, W1, b1, W2T, b2, x, We_flat)


def _k2_kernel(x_ref, Wef_ref, y_ref):
    y_ref[...] = lax.dot_general(
        x_ref[...], Wef_ref[...],
        dimension_numbers=(((1,), (1,)), ((), ())),
        preferred_element_type=jnp.float32)


def _k2(x, We_flat):
    nlo = BH // BB
    return pl.pallas_call(
        _k2_kernel,
        grid=(BH // BB,),
        in_specs=[
            pl.BlockSpec((BB, H), lambda i: (i + nlo, 0)),
            pl.BlockSpec((EO, H), lambda i: (0, 0)),
        ],
        out_specs=pl.BlockSpec((BB, EO), lambda i: (i, 0)),
        out_shape=jax.ShapeDtypeStruct((BH, EO), jnp.float32),
    )(x, We_flat)


def _k3_kernel(g_ref, y_lo_ref, y_hi_ref, beF_ref, outT_ref):
    je = lax.broadcasted_iota(jnp.int32, (E, EO), 1) // OUT
    ee = lax.broadcasted_iota(jnp.int32, (E, EO), 0)
    R = (je == ee).astype(jnp.float32)
    jo = lax.broadcasted_iota(jnp.int32, (OUT, EO), 1) % OUT
    oo = lax.broadcasted_iota(jnp.int32, (OUT, EO), 0)
    ST = (jo == oo).astype(jnp.float32)
    half = pl.program_id(0)
    y = jnp.where(half == 0, y_lo_ref[...], y_hi_ref[...])
    gbig = jnp.dot(g_ref[...], R, preferred_element_type=jnp.float32)
    P = gbig * (y + beF_ref[...])
    outT_ref[...] = lax.dot_general(
        ST, P, dimension_numbers=(((1,), (1,)), ((), ())),
        preferred_element_type=jnp.float32)


def _k3(gates, y_lo, y_hi, beF):
    return pl.pallas_call(
        _k3_kernel,
        grid=(2,),
        in_specs=[
            pl.BlockSpec((BH, E), lambda i: (i, 0)),
            pl.BlockSpec((BH, EO), lambda i: (0, 0)),
            pl.BlockSpec((BH, EO), lambda i: (0, 0)),
            pl.BlockSpec((1, EO), lambda i: (0, 0)),
        ],
        out_specs=pl.BlockSpec((OUT, BH), lambda i: (0, i)),
        out_shape=jax.ShapeDtypeStruct((OUT, B), jnp.float32),
    )(gates, y_lo, y_hi, beF)


def _sc_routing_body(logits_hbm, gates_hbm, lg_v, g_v):
    wid = lax.axis_index("s") * NC + lax.axis_index("c")
    base = wid * TPW
    pltpu.sync_copy(logits_hbm.at[pl.ds(base, TPW)], lg_v)

    lanes = lax.iota(jnp.int32, L)

    def group_body(g, _):
        rowidx = g * L + lanes
        def top2_body(e, carry):
            m0, m1, i0, i1 = carry
            v = plsc.load_gather(lg_v, [rowidx, jnp.full((L,), e, jnp.int32)])
            is0 = v > m0
            is1 = jnp.logical_and(jnp.logical_not(is0), v > m1)
            m1 = jnp.where(is0, m0, jnp.where(is1, v, m1))
            i1 = jnp.where(is0, i0, jnp.where(is1, e, i1))
            m0 = jnp.where(is0, v, m0)
            i0 = jnp.where(is0, e, i0)
            return m0, m1, i0, i1

        init = (jnp.full((L,), -jnp.inf, jnp.float32),
                jnp.full((L,), -jnp.inf, jnp.float32),
                jnp.zeros((L,), jnp.int32),
                jnp.zeros((L,), jnp.int32))
        m0, m1, i0, i1 = lax.fori_loop(0, E, top2_body, init)
        ex = jnp.exp(m1 - m0)
        g0 = 1.0 / (1.0 + ex)
        g1 = ex * g0

        def scatter_body(e, _):
            col = jnp.where(i0 == e, g0, 0.0) + jnp.where(i1 == e, g1, 0.0)
            plsc.store_scatter(g_v, [rowidx, jnp.full((L,), e, jnp.int32)], col)
            return 0

        lax.fori_loop(0, E, scatter_body, 0)
        return 0

    lax.fori_loop(0, NG, group_body, 0)

    pltpu.sync_copy(g_v, gates_hbm.at[pl.ds(base, TPW)])


_sc_routing = functools.partial(
    pl.kernel,
    mesh=plsc.VectorSubcoreMesh(core_axis_name="c", subcore_axis_name="s",
                                num_cores=NC, num_subcores=NS),
    compiler_params=pltpu.CompilerParams(needs_layout_passes=False),
    out_type=jax.ShapeDtypeStruct((B, E), jnp.float32),
    scratch_types=[
        pltpu.VMEM((TPW, E), jnp.float32),
        pltpu.VMEM((TPW, E), jnp.float32),
    ],
)(_sc_routing_body)


def kernel(query_repr, x, W1, b1, W2, b2, We, be):
    We_flat = We.reshape(EO, H)
    logits, y_lo = _k1(query_repr, W1, b1, W2.T, b2, x, We_flat)
    gates = _sc_routing(logits)
    y_hi = _k2(x, We_flat)
    beF = be.reshape(1, EO)
    return _k3(gates, y_lo, y_hi, beF).T


# speedup vs baseline: 1.0334x; 1.0113x over previous
import functools
import jax
import jax.numpy as jnp
from jax import lax
from jax.experimental import pallas as pl
from jax.experimental.pallas import tpu as pltpu
from jax.experimental.pallas import tpu_sc as plsc

B = 2048
H = 1024
HH = 512
E = 16
OUT = 16
EO = E * OUT
BH = B // 2

BB = 512
BQ = BH // (B // BB)
NC = 2
NS = 16
NW = NC * NS
L = 16
TPW = B // NW
NG = TPW // L


def _k1_kernel(q_ref, W1_ref, b1_ref, W2T_ref, b2_ref,
               x_ref, Wef_ref, logits_ref, y_ref):
    h = jnp.maximum(jnp.dot(q_ref[...], W1_ref[...],
                            preferred_element_type=jnp.float32) + b1_ref[...], 0.0)
    logits_ref[...] = lax.dot_general(
        h, W2T_ref[...], dimension_numbers=(((1,), (1,)), ((), ())),
        preferred_element_type=jnp.float32) + b2_ref[...]
    y_ref[...] = lax.dot_general(
        x_ref[...], Wef_ref[...],
        dimension_numbers=(((1,), (1,)), ((), ())),
        preferred_element_type=jnp.float32)


def _k1(q, W1, b1, W2T, b2, x, We_flat):
    return pl.pallas_call(
        _k1_kernel,
        grid=(B // BB,),
        in_specs=[
            pl.BlockSpec((BB, H), lambda i: (i, 0)),
            pl.BlockSpec((H, HH), lambda i: (0, 0)),
            pl.BlockSpec((HH,), lambda i: (0,)),
            pl.BlockSpec((E, HH), lambda i: (0, 0)),
            pl.BlockSpec((E,), lambda i: (0,)),
            pl.BlockSpec((BQ, H), lambda i: (i, 0)),
            pl.BlockSpec((EO, H), lambda i: (0, 0)),
        ],
        out_specs=[
            pl.BlockSpec((BB, E), lambda i: (i, 0)),
            pl.BlockSpec((BQ, EO), lambda i: (i, 0)),
        ],
        out_shape=[
            jax.ShapeDtypeStruct((B, E), jnp.float32),
            jax.ShapeDtypeStruct((BH, EO), jnp.float32),
        ],
    )(q, W1, b1, W2T, b2, x, We_flat)


def _k2_kernel(x_ref, Wef_ref, y_ref):
    y_ref[...] = lax.dot_general(
        x_ref[...], Wef_ref[...],
        dimension_numbers=(((1,), (1,)), ((), ())),
        preferred_element_type=jnp.float32)


def _k2(x, We_flat):
    nlo = BH // BB
    return pl.pallas_call(
        _k2_kernel,
        grid=(BH // BB,),
        in_specs=[
            pl.BlockSpec((BB, H), lambda i: (i + nlo, 0)),
            pl.BlockSpec((EO, H), lambda i: (0, 0)),
        ],
        out_specs=pl.BlockSpec((BB, EO), lambda i: (i, 0)),
        out_shape=jax.ShapeDtypeStruct((BH, EO), jnp.float32),
    )(x, We_flat)


def _k3_kernel(g_ref, y_lo_ref, y_hi_ref, beF_ref, outT_ref):
    je = lax.broadcasted_iota(jnp.int32, (E, EO), 1) // OUT
    ee = lax.broadcasted_iota(jnp.int32, (E, EO), 0)
    R = (je == ee).astype(jnp.float32)
    jo = lax.broadcasted_iota(jnp.int32, (OUT, EO), 1) % OUT
    oo = lax.broadcasted_iota(jnp.int32, (OUT, EO), 0)
    ST = (jo == oo).astype(jnp.float32)
    half = pl.program_id(0)
    y = jnp.where(half == 0, y_lo_ref[...], y_hi_ref[...])
    gbig = jnp.dot(g_ref[...], R, preferred_element_type=jnp.float32)
    P = gbig * (y + beF_ref[...])
    outT_ref[...] = lax.dot_general(
        ST, P, dimension_numbers=(((1,), (1,)), ((), ())),
        preferred_element_type=jnp.float32)


def _k3(gates, y_lo, y_hi, beF):
    return pl.pallas_call(
        _k3_kernel,
        grid=(2,),
        in_specs=[
            pl.BlockSpec((BH, E), lambda i: (i, 0)),
            pl.BlockSpec((BH, EO), lambda i: (0, 0)),
            pl.BlockSpec((BH, EO), lambda i: (0, 0)),
            pl.BlockSpec((1, EO), lambda i: (0, 0)),
        ],
        out_specs=pl.BlockSpec((OUT, BH), lambda i: (0, i)),
        out_shape=jax.ShapeDtypeStruct((OUT, B), jnp.float32),
    )(gates, y_lo, y_hi, beF)


def _sc_routing_body(logits_hbm, gates_hbm, lg_v, g_v):
    wid = lax.axis_index("s") * NC + lax.axis_index("c")
    base = wid * TPW
    pltpu.sync_copy(logits_hbm.at[pl.ds(base, TPW)], lg_v)

    lanes = lax.iota(jnp.int32, L)

    def group_body(g, _):
        rowidx = g * L + lanes
        def top2_body(e, carry):
            m0, m1, i0, i1 = carry
            v = plsc.load_gather(lg_v, [rowidx, jnp.full((L,), e, jnp.int32)])
            is0 = v > m0
            is1 = jnp.logical_and(jnp.logical_not(is0), v > m1)
            m1 = jnp.where(is0, m0, jnp.where(is1, v, m1))
            i1 = jnp.where(is0, i0, jnp.where(is1, e, i1))
            m0 = jnp.where(is0, v, m0)
            i0 = jnp.where(is0, e, i0)
            return m0, m1, i0, i1

        init = (jnp.full((L,), -jnp.inf, jnp.float32),
                jnp.full((L,), -jnp.inf, jnp.float32),
                jnp.zeros((L,), jnp.int32),
                jnp.zeros((L,), jnp.int32))
        m0, m1, i0, i1 = lax.fori_loop(0, E, top2_body, init)
        ex = jnp.exp(m1 - m0)
        g0 = 1.0 / (1.0 + ex)
        g1 = ex * g0

        def scatter_body(e, _):
            col = jnp.where(i0 == e, g0, 0.0) + jnp.where(i1 == e, g1, 0.0)
            plsc.store_scatter(g_v, [rowidx, jnp.full((L,), e, jnp.int32)], col)
            return 0

        lax.fori_loop(0, E, scatter_body, 0)
        return 0

    lax.fori_loop(0, NG, group_body, 0)

    pltpu.sync_copy(g_v, gates_hbm.at[pl.ds(base, TPW)])


_sc_routing = functools.partial(
    pl.kernel,
    mesh=plsc.VectorSubcoreMesh(core_axis_name="c", subcore_axis_name="s",
                                num_cores=NC, num_subcores=NS),
    compiler_params=pltpu.CompilerParams(needs_layout_passes=False),
    out_type=jax.ShapeDtypeStruct((B, E), jnp.float32),
    scratch_types=[
        pltpu.VMEM((TPW, E), jnp.float32),
        pltpu.VMEM((TPW, E), jnp.float32),
    ],
)(_sc_routing_body)


def kernel(query_repr, x, W1, b1, W2, b2, We, be):
    We_flat = We.reshape(EO, H)
    logits, y_lo = _k1(query_repr, W1, b1, W2.T, b2, x, We_flat)
    gates = _sc_routing(logits)
    y_hi = _k2(x, We_flat)
    beF = be.reshape(1, EO)
    return _k3(gates, y_lo, y_hi, beF).T
